# baseline (speedup 1.0000x reference)
import functools

import jax
import jax.numpy as jnp
from jax import lax
from jax.experimental import pallas as pl
from jax.experimental.pallas import tpu as pltpu
from jax.experimental.pallas import tpu_sc as plsc

NC = 2
NS = 16
NW = NC * NS
CH = 128

NUM_STEPS = 6


def _proj_kernel(nf_ref, w_ref, b_ref, o_ref):
    v = jax.nn.relu(
        jnp.dot(nf_ref[...], w_ref[...], preferred_element_type=jnp.float32)
        + b_ref[...]
    )
    o_ref[...] = jnp.concatenate([v, v, v, v], axis=1)


def _project(nf, w, b, blk=1000):
    n, d_in = nf.shape
    d_out = w.shape[1]
    return pl.pallas_call(
        _proj_kernel,
        grid=(n // blk,),
        in_specs=[
            pl.BlockSpec((blk, d_in), lambda i: (i, 0)),
            pl.BlockSpec((d_in, d_out), lambda i: (0, 0)),
            pl.BlockSpec((1, d_out), lambda i: (0, 0)),
        ],
        out_specs=pl.BlockSpec((blk, 4 * d_out), lambda i: (i, 0)),
        out_shape=jax.ShapeDtypeStruct((n, 4 * d_out), jnp.float32),
    )(nf, w, b.reshape(1, d_out))


def _ew_kernel(ef_ref, w1_ref, b1_ref, w2_ref, b2_ref, o_ref):
    z = jax.nn.relu(
        jnp.dot(ef_ref[...], w1_ref[...], preferred_element_type=jnp.float32)
        + b1_ref[...]
    )
    o_ref[...] = (
        jnp.dot(z.astype(jnp.bfloat16), w2_ref[...].astype(jnp.bfloat16),
                preferred_element_type=jnp.float32) + b2_ref[...]
    ).astype(jnp.bfloat16)


def _edge_net(ef, w1, b1, w2, b2, blk=1000):
    e, d_e = ef.shape
    d_hid = w1.shape[1]
    d_oo = w2.shape[1]
    return pl.pallas_call(
        _ew_kernel,
        grid=(e // blk,),
        in_specs=[
            pl.BlockSpec((blk, d_e), lambda i: (i, 0)),
            pl.BlockSpec((d_e, d_hid), lambda i: (0, 0)),
            pl.BlockSpec((1, d_hid), lambda i: (0, 0)),
            pl.BlockSpec((d_hid, d_oo), lambda i: (0, 0)),
            pl.BlockSpec((1, d_oo), lambda i: (0, 0)),
        ],
        out_specs=pl.BlockSpec((blk, d_oo), lambda i: (i, 0)),
        out_shape=jax.ShapeDtypeStruct((e, d_oo), jnp.bfloat16),
    )(ef, w1, b1.reshape(1, d_hid), w2, b2.reshape(1, d_oo))


def _sc_gather(h, src_idx):
    e = src_idx.shape[0]
    d = h.shape[1]
    nchunk = e // CH
    iters = (nchunk + NW - 1) // NW
    mesh = plsc.VectorSubcoreMesh(core_axis_name="c", subcore_axis_name="s")

    @functools.partial(
        pl.kernel,
        mesh=mesh,
        out_type=jax.ShapeDtypeStruct((e, d), jnp.float32),
        scratch_types=[
            pltpu.VMEM((CH,), jnp.int32),
            pltpu.VMEM((CH, d), jnp.float32),
            pltpu.SemaphoreType.DMA,
        ],
    )
    def k(h_hbm, idx_hbm, out_hbm, idx_v, rows_v, sem):
        wid = lax.axis_index("s") * NC + lax.axis_index("c")

        @pl.loop(0, iters)
        def _(t):
            cid = t * NW + wid

            @pl.when(cid < nchunk)
            def _():
                off = pl.multiple_of(cid * CH, CH)
                pltpu.sync_copy(idx_hbm.at[pl.ds(off, CH)], idx_v)
                pltpu.async_copy(h_hbm.at[idx_v], rows_v, sem).wait()
                pltpu.sync_copy(rows_v, out_hbm.at[pl.ds(off, CH)])

    return k(h, src_idx)


def _sc_scatter(m, dst_idx, zeros_n):
    e, d = m.shape
    n = zeros_n.shape[0]
    nchunk = e // CH
    per_core = nchunk // NC
    iters = (per_core + NS - 1) // NS
    rows_per_sub = ((n // NS + 7) // 8) * 8
    last_rows = n - rows_per_sub * (NS - 1)
    mesh = plsc.VectorSubcoreMesh(core_axis_name="c", subcore_axis_name="s")

    @functools.partial(
        pl.kernel,
        mesh=mesh,
        out_type=jax.ShapeDtypeStruct((NC, n, d), jnp.float32),
        scratch_types=[
            pltpu.VMEM((CH,), jnp.int32),
            pltpu.VMEM((CH, d), jnp.float32),
            pltpu.VMEM_SHARED((n, d), jnp.float32),
            pltpu.SemaphoreType.DMA,
        ],
    )
    def k(m_hbm, idx_hbm, z_hbm, out_hbm, idx_v, rows_v, acc_sh, sem):
        c = lax.axis_index("c")
        s = lax.axis_index("s")
        off_r = pl.multiple_of(s * rows_per_sub, 8)

        @pl.when(s < NS - 1)
        def _():
            pltpu.sync_copy(z_hbm.at[pl.ds(off_r, rows_per_sub)],
                            acc_sh.at[pl.ds(off_r, rows_per_sub)])

        @pl.when(s == NS - 1)
        def _():
            pltpu.sync_copy(z_hbm.at[pl.ds(off_r, last_rows)],
                            acc_sh.at[pl.ds(off_r, last_rows)])

        plsc.subcore_barrier()

        @pl.loop(0, iters)
        def _(t):
            lc = t * NS + s

            @pl.when(lc < per_core)
            def _():
                off = pl.multiple_of((c * per_core + lc) * CH, CH)
                pltpu.sync_copy(idx_hbm.at[pl.ds(off, CH)], idx_v)
                pltpu.sync_copy(m_hbm.at[pl.ds(off, CH)], rows_v)
                pltpu.sync_copy(rows_v, acc_sh.at[idx_v], add=True)

        plsc.subcore_barrier()

        @pl.when(s < NS - 1)
        def _():
            pltpu.sync_copy(acc_sh.at[pl.ds(off_r, rows_per_sub)],
                            out_hbm.at[c].at[pl.ds(off_r, rows_per_sub)])

        @pl.when(s == NS - 1)
        def _():
            pltpu.sync_copy(acc_sh.at[pl.ds(off_r, last_rows)],
                            out_hbm.at[c].at[pl.ds(off_r, last_rows)])

    return k(m, dst_idx, zeros_n)


def _msg_kernel(hs_ref, w_ref, m_ref):
    hs = hs_ref[...][:, :32].astype(jnp.bfloat16).astype(jnp.float32)
    w = w_ref[...].astype(jnp.float32)
    blk, d = hs.shape
    w3 = w.reshape(blk, d, d)
    m = jnp.sum(hs[:, :, None] * w3, axis=1)
    m_ref[...] = jnp.concatenate([m, m, m, m], axis=1)


def _messages(hs, w_edge, blk=1000):
    e = hs.shape[0]
    d = 32
    return pl.pallas_call(
        _msg_kernel,
        grid=(e // blk,),
        in_specs=[
            pl.BlockSpec((blk, 4 * d), lambda i: (i, 0)),
            pl.BlockSpec((blk, d * d), lambda i: (i, 0)),
        ],
        out_specs=pl.BlockSpec((blk, 4 * d), lambda i: (i, 0)),
        out_shape=jax.ShapeDtypeStruct((e, 4 * d), jnp.float32),
    )(hs, w_edge)


def _gru_kernel(agg_ref, h_ref, wih_ref, whh_ref, bih_ref, bhh_ref, cb_ref, o_ref):
    d = 32
    agg = (agg_ref[0][:, :d] + agg_ref[1][:, :d] + cb_ref[...]).astype(
        jnp.bfloat16).astype(jnp.float32)
    x = jax.nn.relu(agg)
    h = h_ref[...][:, :d]
    gi = jnp.dot(x, wih_ref[...], preferred_element_type=jnp.float32) + bih_ref[...]
    gh = jnp.dot(h, whh_ref[...], preferred_element_type=jnp.float32) + bhh_ref[...]
    r = jax.nn.sigmoid(gi[:, :d] + gh[:, :d])
    z = jax.nn.sigmoid(gi[:, d:2 * d] + gh[:, d:2 * d])
    n = jnp.tanh(gi[:, 2 * d:] + r * gh[:, 2 * d:])
    hn = (1.0 - z) * n + z * h
    o_ref[...] = jnp.concatenate([hn, hn, hn, hn], axis=1)


def _gru(agg2, h, wih_t, whh_t, bih, bhh, cb, blk=1000):
    n = h.shape[0]
    d = 32
    return pl.pallas_call(
        _gru_kernel,
        grid=(n // blk,),
        in_specs=[
            pl.BlockSpec((NC, blk, 4 * d), lambda i: (0, i, 0)),
            pl.BlockSpec((blk, 4 * d), lambda i: (i, 0)),
            pl.BlockSpec((d, 3 * d), lambda i: (0, 0)),
            pl.BlockSpec((d, 3 * d), lambda i: (0, 0)),
            pl.BlockSpec((1, 3 * d), lambda i: (0, 0)),
            pl.BlockSpec((1, 3 * d), lambda i: (0, 0)),
            pl.BlockSpec((1, d), lambda i: (0, 0)),
        ],
        out_specs=pl.BlockSpec((blk, 4 * d), lambda i: (i, 0)),
        out_shape=jax.ShapeDtypeStruct((n, 4 * d), jnp.float32),
    )(agg2, h, wih_t, whh_t, bih.reshape(1, 3 * d), bhh.reshape(1, 3 * d),
      cb.reshape(1, d))


def kernel(node_feats, edge_feats, edge_index, W_proj, b_proj, W_e1, b_e1,
           W_e2, b_e2, conv_bias, W_ih, W_hh, b_ih, b_hh):
    n = node_feats.shape[0]
    d = W_proj.shape[1]
    src = edge_index[0].astype(jnp.int32)
    dst = edge_index[1].astype(jnp.int32)
    zeros_n = jnp.zeros((n, 4 * d), jnp.float32)
    wih_t = W_ih.T
    whh_t = W_hh.T

    h = _project(node_feats, W_proj, b_proj)
    w_edge = _edge_net(edge_feats, W_e1, b_e1, W_e2, b_e2)

    for _ in range(NUM_STEPS):
        hs = _sc_gather(h, src)
        m = _messages(hs, w_edge)
        agg2 = _sc_scatter(m, dst, zeros_n)
        h = _gru(agg2, h, wih_t, whh_t, b_ih, b_hh, conv_bias)
    return h[:, :d]


# speedup vs baseline: 2.6422x; 2.6422x over previous
import functools

import jax
import jax.numpy as jnp
from jax import lax
from jax.experimental import pallas as pl
from jax.experimental.pallas import tpu as pltpu
from jax.experimental.pallas import tpu_sc as plsc

NC = 2
NS = 16
NW = NC * NS
CH = 128

NUM_STEPS = 6


def _proj_kernel(nf_ref, w_ref, b_ref, o_ref):
    v = jax.nn.relu(
        jnp.dot(nf_ref[...], w_ref[...], preferred_element_type=jnp.float32)
        + b_ref[...]
    )
    o_ref[...] = jnp.concatenate([v, v, v, v], axis=1)


def _project(nf, w, b, blk=1000):
    n, d_in = nf.shape
    d_out = w.shape[1]
    return pl.pallas_call(
        _proj_kernel,
        grid=(n // blk,),
        in_specs=[
            pl.BlockSpec((blk, d_in), lambda i: (i, 0)),
            pl.BlockSpec((d_in, d_out), lambda i: (0, 0)),
            pl.BlockSpec((1, d_out), lambda i: (0, 0)),
        ],
        out_specs=pl.BlockSpec((blk, 4 * d_out), lambda i: (i, 0)),
        out_shape=jax.ShapeDtypeStruct((n, 4 * d_out), jnp.float32),
    )(nf, w, b.reshape(1, d_out))


def _ew_kernel(ef_ref, w1_ref, b1_ref, w2_ref, b2_ref, o_ref):
    z = jax.nn.relu(
        jnp.dot(ef_ref[...], w1_ref[...], preferred_element_type=jnp.float32)
        + b1_ref[...]
    )
    o_ref[...] = (
        jnp.dot(z.astype(jnp.bfloat16), w2_ref[...].astype(jnp.bfloat16),
                preferred_element_type=jnp.float32) + b2_ref[...]
    ).astype(jnp.bfloat16)


def _edge_net(ef, w1, b1, w2, b2, blk=1000):
    e, d_e = ef.shape
    d_hid = w1.shape[1]
    d_oo = w2.shape[1]
    return pl.pallas_call(
        _ew_kernel,
        grid=(e // blk,),
        in_specs=[
            pl.BlockSpec((blk, d_e), lambda i: (i, 0)),
            pl.BlockSpec((d_e, d_hid), lambda i: (0, 0)),
            pl.BlockSpec((1, d_hid), lambda i: (0, 0)),
            pl.BlockSpec((d_hid, d_oo), lambda i: (0, 0)),
            pl.BlockSpec((1, d_oo), lambda i: (0, 0)),
        ],
        out_specs=pl.BlockSpec((blk, d_oo), lambda i: (i, 0)),
        out_shape=jax.ShapeDtypeStruct((e, d_oo), jnp.bfloat16),
    )(ef, w1, b1.reshape(1, d_hid), w2, b2.reshape(1, d_oo))


def _sc_gather(h, src_idx):
    e = src_idx.shape[0]
    d = h.shape[1]
    nchunk = e // CH
    iters = (nchunk + NW - 1) // NW
    mesh = plsc.VectorSubcoreMesh(core_axis_name="c", subcore_axis_name="s")

    @functools.partial(
        pl.kernel,
        mesh=mesh,
        out_type=jax.ShapeDtypeStruct((e, d), jnp.float32),
        scratch_types=[
            pltpu.VMEM((CH,), jnp.int32),
            pltpu.VMEM((CH, d), jnp.float32),
            pltpu.SemaphoreType.DMA,
        ],
    )
    def k(h_hbm, idx_hbm, out_hbm, idx_v, rows_v, sem):
        wid = lax.axis_index("s") * NC + lax.axis_index("c")

        @pl.loop(0, iters)
        def _(t):
            cid = t * NW + wid

            @pl.when(cid < nchunk)
            def _():
                off = pl.multiple_of(cid * CH, CH)
                pltpu.sync_copy(idx_hbm.at[pl.ds(off, CH)], idx_v)
                pltpu.async_copy(h_hbm.at[idx_v], rows_v, sem).wait()
                pltpu.sync_copy(rows_v, out_hbm.at[pl.ds(off, CH)])

    return k(h, src_idx)


def _sc_scatter(m, dst_idx, zeros_n):
    e, d = m.shape
    n = zeros_n.shape[0]
    nchunk = e // CH
    per_core = nchunk // NC
    iters = (per_core + NS - 1) // NS
    rows_per_sub = ((n // NS + 7) // 8) * 8
    last_rows = n - rows_per_sub * (NS - 1)
    mesh = plsc.VectorSubcoreMesh(core_axis_name="c", subcore_axis_name="s")

    @functools.partial(
        pl.kernel,
        mesh=mesh,
        out_type=jax.ShapeDtypeStruct((NC, n, d), jnp.float32),
        scratch_types=[
            pltpu.VMEM((CH,), jnp.int32),
            pltpu.VMEM((CH, d), jnp.float32),
            pltpu.VMEM_SHARED((n, d), jnp.float32),
            pltpu.SemaphoreType.DMA,
        ],
    )
    def k(m_hbm, idx_hbm, z_hbm, out_hbm, idx_v, rows_v, acc_sh, sem):
        c = lax.axis_index("c")
        s = lax.axis_index("s")
        off_r = pl.multiple_of(s * rows_per_sub, 8)

        @pl.when(s < NS - 1)
        def _():
            pltpu.sync_copy(z_hbm.at[pl.ds(off_r, rows_per_sub)],
                            acc_sh.at[pl.ds(off_r, rows_per_sub)])

        @pl.when(s == NS - 1)
        def _():
            pltpu.sync_copy(z_hbm.at[pl.ds(off_r, last_rows)],
                            acc_sh.at[pl.ds(off_r, last_rows)])

        plsc.subcore_barrier()

        @pl.loop(0, iters)
        def _(t):
            lc = t * NS + s

            @pl.when(lc < per_core)
            def _():
                off = pl.multiple_of((c * per_core + lc) * CH, CH)
                pltpu.sync_copy(idx_hbm.at[pl.ds(off, CH)], idx_v)
                pltpu.sync_copy(m_hbm.at[pl.ds(off, CH)], rows_v)
                pltpu.sync_copy(rows_v, acc_sh.at[idx_v], add=True)

        plsc.subcore_barrier()

        @pl.when(s < NS - 1)
        def _():
            pltpu.sync_copy(acc_sh.at[pl.ds(off_r, rows_per_sub)],
                            out_hbm.at[c].at[pl.ds(off_r, rows_per_sub)])

        @pl.when(s == NS - 1)
        def _():
            pltpu.sync_copy(acc_sh.at[pl.ds(off_r, last_rows)],
                            out_hbm.at[c].at[pl.ds(off_r, last_rows)])

    return k(m, dst_idx, zeros_n)


def _msg_kernel(hs_ref, w_ref, r_ref, m_ref):
    hsq = hs_ref[...][:, :32].astype(jnp.bfloat16)
    rep = jnp.dot(hsq, r_ref[...], preferred_element_type=jnp.float32)
    mw = rep * w_ref[...].astype(jnp.float32)
    f = mw[:, :512] + mw[:, 512:]
    f = f[:, :256] + f[:, 256:]
    f = f[:, :128] + f[:, 128:]
    f = f[:, :64] + f[:, 64:]
    m = f[:, :32] + f[:, 32:]
    m_ref[...] = jnp.concatenate([m, m, m, m], axis=1)


def _messages(hs, w_edge, rep_mat, blk=1000):
    e = hs.shape[0]
    d = 32
    return pl.pallas_call(
        _msg_kernel,
        grid=(e // blk,),
        in_specs=[
            pl.BlockSpec((blk, 4 * d), lambda i: (i, 0)),
            pl.BlockSpec((blk, d * d), lambda i: (i, 0)),
            pl.BlockSpec((d, d * d), lambda i: (0, 0)),
        ],
        out_specs=pl.BlockSpec((blk, 4 * d), lambda i: (i, 0)),
        out_shape=jax.ShapeDtypeStruct((e, 4 * d), jnp.float32),
    )(hs, w_edge, rep_mat)


def _gru_kernel(agg_ref, h_ref, wih_ref, whh_ref, bih_ref, bhh_ref, cb_ref, o_ref):
    d = 32
    agg = (agg_ref[0][:, :d] + agg_ref[1][:, :d] + cb_ref[...]).astype(
        jnp.bfloat16).astype(jnp.float32)
    x = jax.nn.relu(agg)
    h = h_ref[...][:, :d]
    gi = jnp.dot(x, wih_ref[...], preferred_element_type=jnp.float32) + bih_ref[...]
    gh = jnp.dot(h, whh_ref[...], preferred_element_type=jnp.float32) + bhh_ref[...]
    r = jax.nn.sigmoid(gi[:, :d] + gh[:, :d])
    z = jax.nn.sigmoid(gi[:, d:2 * d] + gh[:, d:2 * d])
    n = jnp.tanh(gi[:, 2 * d:] + r * gh[:, 2 * d:])
    hn = (1.0 - z) * n + z * h
    o_ref[...] = jnp.concatenate([hn, hn, hn, hn], axis=1)


def _gru(agg2, h, wih_t, whh_t, bih, bhh, cb, blk=1000):
    n = h.shape[0]
    d = 32
    return pl.pallas_call(
        _gru_kernel,
        grid=(n // blk,),
        in_specs=[
            pl.BlockSpec((NC, blk, 4 * d), lambda i: (0, i, 0)),
            pl.BlockSpec((blk, 4 * d), lambda i: (i, 0)),
            pl.BlockSpec((d, 3 * d), lambda i: (0, 0)),
            pl.BlockSpec((d, 3 * d), lambda i: (0, 0)),
            pl.BlockSpec((1, 3 * d), lambda i: (0, 0)),
            pl.BlockSpec((1, 3 * d), lambda i: (0, 0)),
            pl.BlockSpec((1, d), lambda i: (0, 0)),
        ],
        out_specs=pl.BlockSpec((blk, 4 * d), lambda i: (i, 0)),
        out_shape=jax.ShapeDtypeStruct((n, 4 * d), jnp.float32),
    )(agg2, h, wih_t, whh_t, bih.reshape(1, 3 * d), bhh.reshape(1, 3 * d),
      cb.reshape(1, d))


def kernel(node_feats, edge_feats, edge_index, W_proj, b_proj, W_e1, b_e1,
           W_e2, b_e2, conv_bias, W_ih, W_hh, b_ih, b_hh):
    n = node_feats.shape[0]
    d = W_proj.shape[1]
    src = edge_index[0].astype(jnp.int32)
    dst = edge_index[1].astype(jnp.int32)
    zeros_n = jnp.zeros((n, 4 * d), jnp.float32)
    wih_t = W_ih.T
    whh_t = W_hh.T
    rep_mat = jnp.kron(jnp.eye(d, dtype=jnp.float32),
                       jnp.ones((1, d), jnp.float32)).astype(jnp.bfloat16)

    h = _project(node_feats, W_proj, b_proj)
    w_edge = _edge_net(edge_feats, W_e1, b_e1, W_e2, b_e2)

    for _ in range(NUM_STEPS):
        hs = _sc_gather(h, src)
        m = _messages(hs, w_edge, rep_mat)
        agg2 = _sc_scatter(m, dst, zeros_n)
        h = _gru(agg2, h, wih_t, whh_t, b_ih, b_hh, conv_bias)
    return h[:, :d]
